# Initial kernel scaffold; baseline (speedup 1.0000x reference)
import jax
import jax.numpy as jnp
from jax import lax
from jax.experimental import pallas as pl
from jax.experimental.pallas import tpu as pltpu
from jax.experimental.pallas import tpu_sc as plsc

N_NODES = 10000
N_FEATS = 9
N_GRAPHS = 512
EMB = 128
ROWS = N_FEATS * EMB

NT = 16
NPT = 640
NPAD = NT * NPT
NW = NPT // 16
RPT = ROWS // NT
ACC = 544
GPT = N_GRAPHS // NT


def _sc_kernel(xf_hbm, bi_hbm, tab_hbm, w_hbm, b_hbm, out_hbm,
               x_v, bi_v, tab_v, w_v, tw72_v, tw_v, s_v, sums_v, cnts_v,
               mg_s, mg_c, out_v, b_v, tw_sh, part_sh):
    sid = lax.axis_index("s")
    iota = lax.iota(jnp.int32, 16)

    pltpu.sync_copy(w_hbm, w_v)
    pltpu.sync_copy(tab_hbm.at[pl.ds(sid * (RPT * EMB), RPT * EMB)], tab_v)

    def row_body(r, _):
        acc = jnp.zeros((16,), jnp.float32)
        for k in range(EMB // 16):
            acc = acc + tab_v[pl.ds(r * EMB + k * 16, 16)] * w_v[pl.ds(k * 16, 16)]
        tw72_v[r] = jnp.sum(acc, axis=0)
        return 0

    lax.fori_loop(0, RPT, row_body, 0)
    pltpu.sync_copy(tw72_v, tw_sh.at[pl.ds(sid * RPT, RPT)])
    plsc.subcore_barrier()
    pltpu.sync_copy(tw_sh, tw_v)

    pltpu.sync_copy(xf_hbm.at[pl.ds(sid * (NPT * N_FEATS), NPT * N_FEATS)], x_v)
    pltpu.sync_copy(bi_hbm.at[pl.ds(sid * NPT, NPT)], bi_v)
    pltpu.sync_copy(b_hbm, b_v)

    zeros16 = jnp.zeros((16,), jnp.float32)
    for z in range(ACC // 16):
        sums_v[pl.ds(z * 16, 16)] = zeros16
        cnts_v[pl.ds(z * 16, 16)] = zeros16

    def win_body(w, _):
        base = w * (16 * N_FEATS)
        acc = jnp.zeros((16,), jnp.float32)
        for f in range(N_FEATS):
            xi = plsc.load_gather(x_v, [iota * N_FEATS + (base + f)])
            acc = acc + plsc.load_gather(tw_v, [xi + f * EMB])
        s_v[pl.ds(w * 16, 16)] = acc
        return 0

    lax.fori_loop(0, NW, win_body, 0)

    def seg_body(j, _):
        g = bi_v[j]
        sums_v[g] = sums_v[g] + s_v[j]
        cnts_v[g] = cnts_v[g] + 1.0
        return 0

    lax.fori_loop(0, NPT, seg_body, 0)

    pltpu.sync_copy(sums_v, part_sh.at[sid, pl.ds(0, ACC)])
    pltpu.sync_copy(cnts_v, part_sh.at[sid, pl.ds(ACC, ACC)])
    plsc.subcore_barrier()

    g0 = sid * GPT
    pltpu.sync_copy(part_sh.at[:, pl.ds(g0, GPT)], mg_s)
    pltpu.sync_copy(part_sh.at[:, pl.ds(ACC + g0, GPT)], mg_c)

    bb = b_v[pl.ds(0, 16)]
    for half in range(GPT // 16):
        tot = jnp.zeros((16,), jnp.float32)
        cnt = jnp.zeros((16,), jnp.float32)
        for t in range(NT):
            tot = tot + mg_s[t, pl.ds(half * 16, 16)]
            cnt = cnt + mg_c[t, pl.ds(half * 16, 16)]
        z = tot / jnp.maximum(cnt, 1.0) + bb
        out_v[pl.ds(half * 16, 16)] = 1.0 / (1.0 + jnp.exp(-z))

    pltpu.sync_copy(out_v, out_hbm.at[pl.ds(g0, GPT)])


def kernel(x, edge_index, batch_idx, atom_tables, W, b):
    xf = jnp.pad(x.astype(jnp.int32).reshape(-1), (0, NPAD * N_FEATS - N_NODES * N_FEATS))
    bi = jnp.pad(batch_idx.astype(jnp.int32), (0, NPAD - N_NODES),
                 constant_values=N_GRAPHS)
    tab = atom_tables.reshape(ROWS * EMB)
    wf = W.reshape(EMB)
    b16 = jnp.broadcast_to(b.reshape(-1)[:1], (16,)).astype(jnp.float32)

    mesh = plsc.VectorSubcoreMesh(core_axis_name="c", subcore_axis_name="s",
                                  num_cores=1)
    out = pl.kernel(
        _sc_kernel,
        out_type=jax.ShapeDtypeStruct((N_GRAPHS,), jnp.float32),
        mesh=mesh,
        scratch_types=[
            pltpu.VMEM((NPT * N_FEATS,), jnp.int32),
            pltpu.VMEM((NPT,), jnp.int32),
            pltpu.VMEM((RPT * EMB,), jnp.float32),
            pltpu.VMEM((EMB,), jnp.float32),
            pltpu.VMEM((RPT,), jnp.float32),
            pltpu.VMEM((ROWS,), jnp.float32),
            pltpu.VMEM((NPT,), jnp.float32),
            pltpu.VMEM((ACC,), jnp.float32),
            pltpu.VMEM((ACC,), jnp.float32),
            pltpu.VMEM((NT, GPT), jnp.float32),
            pltpu.VMEM((NT, GPT), jnp.float32),
            pltpu.VMEM((GPT,), jnp.float32),
            pltpu.VMEM((16,), jnp.float32),
            pltpu.VMEM_SHARED((ROWS,), jnp.float32),
            pltpu.VMEM_SHARED((NT, 2 * ACC), jnp.float32),
        ],
    )(xf, bi, tab, wf, b16)
    return out.reshape(N_GRAPHS, 1)


# speedup vs baseline: 34.4607x; 34.4607x over previous
import jax
import jax.numpy as jnp
from jax import lax
from jax.experimental import pallas as pl
from jax.experimental.pallas import tpu as pltpu
from jax.experimental.pallas import tpu_sc as plsc

N_NODES = 10000
N_FEATS = 9
N_GRAPHS = 512
EMB = 128
ROWS = N_FEATS * EMB

NT = 16
NPT = 640
NPAD = NT * NPT
NW = NPT // 16
ACC = 544
GPT = N_GRAPHS // NT


def _tw_matvec(tab_ref, w_ref, out_ref):
    out_ref[...] = jnp.dot(tab_ref[...], w_ref[...],
                           preferred_element_type=jnp.float32)


def _sc_kernel(xf_hbm, bi_hbm, tw_hbm, b_hbm, out_hbm,
               x_v, bi_v, tw_v, sums_v, cnts_v, st_i, st_f,
               mg_s, mg_c, out_v, b_v, part_sh):
    sid = lax.axis_index("s")
    lane = lax.iota(jnp.int32, 16)

    pltpu.sync_copy(tw_hbm, tw_v)
    pltpu.sync_copy(xf_hbm.at[pl.ds(sid * (NPT * N_FEATS), NPT * N_FEATS)], x_v)
    pltpu.sync_copy(bi_hbm.at[pl.ds(sid * NPT, NPT)], bi_v)
    pltpu.sync_copy(b_hbm, b_v)

    zeros16 = jnp.zeros((16,), jnp.float32)
    for z in range(ACC // 16):
        sums_v[pl.ds(z * 16, 16)] = zeros16
        cnts_v[pl.ds(z * 16, 16)] = zeros16

    def win_body(w, _):
        base = w * (16 * N_FEATS)
        s = jnp.zeros((16,), jnp.float32)
        for f in range(N_FEATS):
            xi = plsc.load_gather(x_v, [lane * N_FEATS + (base + f)])
            s = s + plsc.load_gather(tw_v, [xi + f * EMB])
        bidx = bi_v[pl.ds(w * 16, 16)]

        st_i[...] = bidx
        bnext = plsc.load_gather(st_i, [jnp.minimum(lane + 1, 15)])
        is_last = (lane == 15) | (bidx != bnext)

        csum = plsc.cumsum(s)
        cand = jnp.where(is_last, lane, -1)
        cm = plsc.cummax(cand)
        st_i[...] = cm
        pb = plsc.load_gather(st_i, [jnp.maximum(lane - 1, 0)])
        pb = jnp.where(lane == 0, -1, pb)

        st_f[...] = csum
        pcs = plsc.load_gather(st_f, [jnp.maximum(pb, 0)])
        pcs = jnp.where(pb < 0, 0.0, pcs)

        seg_sum = csum - pcs
        seg_cnt = (lane - pb).astype(jnp.float32)
        plsc.addupdate_scatter(sums_v, [bidx], seg_sum, mask=is_last)
        plsc.addupdate_scatter(cnts_v, [bidx], seg_cnt, mask=is_last)
        return 0

    lax.fori_loop(0, NW, win_body, 0)

    pltpu.sync_copy(sums_v, part_sh.at[pl.ds(sid * (2 * ACC), ACC)])
    pltpu.sync_copy(cnts_v, part_sh.at[pl.ds(sid * (2 * ACC) + ACC, ACC)])
    plsc.subcore_barrier()

    g0 = sid * GPT
    for t in range(NT):
        pltpu.sync_copy(part_sh.at[pl.ds(t * (2 * ACC) + g0, GPT)],
                        mg_s.at[pl.ds(t * GPT, GPT)])
        pltpu.sync_copy(part_sh.at[pl.ds(t * (2 * ACC) + ACC + g0, GPT)],
                        mg_c.at[pl.ds(t * GPT, GPT)])

    bb = b_v[pl.ds(0, 16)]
    for half in range(GPT // 16):
        tot = jnp.zeros((16,), jnp.float32)
        cnt = jnp.zeros((16,), jnp.float32)
        for t in range(NT):
            tot = tot + mg_s[pl.ds(t * GPT + half * 16, 16)]
            cnt = cnt + mg_c[pl.ds(t * GPT + half * 16, 16)]
        z = tot / jnp.maximum(cnt, 1.0) + bb
        out_v[pl.ds(half * 16, 16)] = 1.0 / (1.0 + jnp.exp(-z))

    pltpu.sync_copy(out_v, out_hbm.at[pl.ds(g0, GPT)])


def kernel(x, edge_index, batch_idx, atom_tables, W, b):
    xf = jnp.pad(x.astype(jnp.int32).reshape(-1), (0, (NPAD - N_NODES) * N_FEATS))
    bi = jnp.pad(batch_idx.astype(jnp.int32), (0, NPAD - N_NODES),
                 constant_values=N_GRAPHS)
    tab = atom_tables.reshape(ROWS, EMB)
    b16 = jnp.broadcast_to(b.reshape(-1)[:1], (16,)).astype(jnp.float32)

    tw = pl.pallas_call(
        _tw_matvec,
        out_shape=jax.ShapeDtypeStruct((ROWS, 1), jnp.float32),
    )(tab, W.astype(jnp.float32)).reshape(ROWS)

    mesh = plsc.VectorSubcoreMesh(core_axis_name="c", subcore_axis_name="s",
                                  num_cores=1, num_subcores=NT)
    out = pl.kernel(
        _sc_kernel,
        out_type=jax.ShapeDtypeStruct((N_GRAPHS,), jnp.float32),
        mesh=mesh,
        compiler_params=pltpu.CompilerParams(needs_layout_passes=False),
        scratch_types=[
            pltpu.VMEM((NPT * N_FEATS,), jnp.int32),
            pltpu.VMEM((NPT,), jnp.int32),
            pltpu.VMEM((ROWS,), jnp.float32),
            pltpu.VMEM((ACC,), jnp.float32),
            pltpu.VMEM((ACC,), jnp.float32),
            pltpu.VMEM((16,), jnp.int32),
            pltpu.VMEM((16,), jnp.float32),
            pltpu.VMEM((NT * GPT,), jnp.float32),
            pltpu.VMEM((NT * GPT,), jnp.float32),
            pltpu.VMEM((GPT,), jnp.float32),
            pltpu.VMEM((16,), jnp.float32),
            pltpu.VMEM_SHARED((NT * 2 * ACC,), jnp.float32),
        ],
    )(xf, bi, tw, b16)
    return out.reshape(N_GRAPHS, 1)


# speedup vs baseline: 40.4290x; 1.1732x over previous
import jax
import jax.numpy as jnp
from jax import lax
from jax.experimental import pallas as pl
from jax.experimental.pallas import tpu as pltpu
from jax.experimental.pallas import tpu_sc as plsc

N_NODES = 10000
N_FEATS = 9
N_GRAPHS = 512
EMB = 128
ROWS = N_FEATS * EMB

NT = 16
NPT = 640
NPAD = NT * NPT
NW = NPT // 16
ACC = 544
GPT = N_GRAPHS // NT


def _tw_matvec(tab_ref, w_ref, out_ref):
    out_ref[...] = jnp.dot(tab_ref[...], w_ref[...],
                           preferred_element_type=jnp.float32)


def _sc_kernel(xf_hbm, bi_hbm, tw_hbm, b_hbm, out_hbm,
               x_v, bi_v, tw_v, sums_v, cnts_v, st_i, st_f,
               mg_s, mg_c, out_v, b_v, part_sh):
    sid = lax.axis_index("s")
    lane = lax.iota(jnp.int32, 16)

    pltpu.sync_copy(tw_hbm, tw_v)
    pltpu.sync_copy(xf_hbm.at[pl.ds(sid * (NPT * N_FEATS), NPT * N_FEATS)], x_v)
    pltpu.sync_copy(bi_hbm.at[pl.ds(sid * NPT, NPT)], bi_v)
    pltpu.sync_copy(b_hbm, b_v)

    zeros16 = jnp.zeros((16,), jnp.float32)
    for z in range(ACC // 16):
        sums_v[pl.ds(z * 16, 16)] = zeros16
        cnts_v[pl.ds(z * 16, 16)] = zeros16

    def win_body(w, _):
        s = jnp.zeros((16,), jnp.float32)
        for f in range(N_FEATS):
            xi = x_v[pl.ds(f * NPT + w * 16, 16)]
            s = s + plsc.load_gather(tw_v, [xi + f * EMB])
        bidx = bi_v[pl.ds(w * 16, 16)]

        st_i[...] = bidx
        bnext = plsc.load_gather(st_i, [jnp.minimum(lane + 1, 15)])
        is_last = (lane == 15) | (bidx != bnext)

        csum = plsc.cumsum(s)
        cand = jnp.where(is_last, lane, -1)
        cm = plsc.cummax(cand)
        st_i[...] = cm
        pb = plsc.load_gather(st_i, [jnp.maximum(lane - 1, 0)])
        pb = jnp.where(lane == 0, -1, pb)

        st_f[...] = csum
        pcs = plsc.load_gather(st_f, [jnp.maximum(pb, 0)])
        pcs = jnp.where(pb < 0, 0.0, pcs)

        seg_sum = csum - pcs
        seg_cnt = (lane - pb).astype(jnp.float32)
        plsc.addupdate_scatter(sums_v, [bidx], seg_sum, mask=is_last)
        plsc.addupdate_scatter(cnts_v, [bidx], seg_cnt, mask=is_last)
        return 0

    lax.fori_loop(0, NW, win_body, 0)

    pltpu.sync_copy(sums_v, part_sh.at[pl.ds(sid * (2 * ACC), ACC)])
    pltpu.sync_copy(cnts_v, part_sh.at[pl.ds(sid * (2 * ACC) + ACC, ACC)])
    plsc.subcore_barrier()

    g0 = sid * GPT
    for t in range(NT):
        pltpu.sync_copy(part_sh.at[pl.ds(t * (2 * ACC) + g0, GPT)],
                        mg_s.at[pl.ds(t * GPT, GPT)])
        pltpu.sync_copy(part_sh.at[pl.ds(t * (2 * ACC) + ACC + g0, GPT)],
                        mg_c.at[pl.ds(t * GPT, GPT)])

    bb = b_v[pl.ds(0, 16)]
    for half in range(GPT // 16):
        tot = jnp.zeros((16,), jnp.float32)
        cnt = jnp.zeros((16,), jnp.float32)
        for t in range(NT):
            tot = tot + mg_s[pl.ds(t * GPT + half * 16, 16)]
            cnt = cnt + mg_c[pl.ds(t * GPT + half * 16, 16)]
        z = tot / jnp.maximum(cnt, 1.0) + bb
        out_v[pl.ds(half * 16, 16)] = 1.0 / (1.0 + jnp.exp(-z))

    pltpu.sync_copy(out_v, out_hbm.at[pl.ds(g0, GPT)])


def kernel(x, edge_index, batch_idx, atom_tables, W, b):
    xp = jnp.pad(x.astype(jnp.int32), ((0, NPAD - N_NODES), (0, 0)))
    xf = xp.reshape(NT, NPT, N_FEATS).transpose(0, 2, 1).reshape(-1)
    bi = jnp.pad(batch_idx.astype(jnp.int32), (0, NPAD - N_NODES),
                 constant_values=N_GRAPHS)
    tab = atom_tables.reshape(ROWS, EMB)
    b16 = jnp.broadcast_to(b.reshape(-1)[:1], (16,)).astype(jnp.float32)

    tw = pl.pallas_call(
        _tw_matvec,
        out_shape=jax.ShapeDtypeStruct((ROWS, 1), jnp.float32),
    )(tab, W.astype(jnp.float32)).reshape(ROWS)

    mesh = plsc.VectorSubcoreMesh(core_axis_name="c", subcore_axis_name="s",
                                  num_cores=1, num_subcores=NT)
    out = pl.kernel(
        _sc_kernel,
        out_type=jax.ShapeDtypeStruct((N_GRAPHS,), jnp.float32),
        mesh=mesh,
        compiler_params=pltpu.CompilerParams(needs_layout_passes=False),
        scratch_types=[
            pltpu.VMEM((NPT * N_FEATS,), jnp.int32),
            pltpu.VMEM((NPT,), jnp.int32),
            pltpu.VMEM((ROWS,), jnp.float32),
            pltpu.VMEM((ACC,), jnp.float32),
            pltpu.VMEM((ACC,), jnp.float32),
            pltpu.VMEM((16,), jnp.int32),
            pltpu.VMEM((16,), jnp.float32),
            pltpu.VMEM((NT * GPT,), jnp.float32),
            pltpu.VMEM((NT * GPT,), jnp.float32),
            pltpu.VMEM((GPT,), jnp.float32),
            pltpu.VMEM((16,), jnp.float32),
            pltpu.VMEM_SHARED((NT * 2 * ACC,), jnp.float32),
        ],
    )(xf, bi, tw, b16)
    return out.reshape(N_GRAPHS, 1)
